# baseline (speedup 1.0000x reference)
import jax
import jax.numpy as jnp
from jax.experimental import pallas as pl

_LOG2E = 1.4426950408889634


def _ce_body(y_pred_ref, y_true_ref, out_ref):
    b = pl.program_id(0)
    C, Hb, W = y_pred_ref.shape[1:]
    P = 8

    @pl.when(b == 0)
    def _():
        out_ref[...] = jnp.zeros_like(out_ref)

    for w0 in range(0, W, 128):
        cols = pl.ds(w0, 128)
        acc = jnp.zeros((1, 128), jnp.float32)
        for p in range(Hb // P):
            rows = pl.ds(p * P, P)
            y = y_true_ref[0, rows, cols]
            xs = [y_pred_ref[0, c, rows, cols] for c in range(C)]
            m = xs[0]
            sel = xs[0]
            for c in range(1, C):
                m = jnp.maximum(m, xs[c])
                sel = jnp.where(y == c, xs[c], sel)
            ml = m * _LOG2E
            s = jnp.exp2(xs[0] * _LOG2E - ml)
            for c in range(1, C):
                s += jnp.exp2(xs[c] * _LOG2E - ml)
            acc += jnp.sum(m + jnp.log(s) - sel, axis=0, keepdims=True)
        out_ref[:, cols] += acc


def kernel(y_pred, y_true):
    B, C, H, W = y_pred.shape
    out = pl.pallas_call(
        _ce_body,
        grid=(B,),
        in_specs=[
            pl.BlockSpec((1, C, H, W), lambda b: (b, 0, 0, 0)),
            pl.BlockSpec((1, H, W), lambda b: (b, 0, 0)),
        ],
        out_specs=pl.BlockSpec((1, W), lambda b: (0, 0)),
        out_shape=jax.ShapeDtypeStruct((1, W), jnp.float32),
    )(y_pred, y_true)
    return jnp.sum(out) / (B * H * W)
